# baseline (speedup 1.0000x reference)
import functools

import jax
import jax.numpy as jnp
from jax import lax
from jax.experimental import pallas as pl
from jax.experimental.pallas import tpu as pltpu
from jax.experimental.pallas import tpu_sc as plsc

_EMB = 1024
_E = 8
_TOP = 2
_W_IMP = 0.01
_N = 2048
_BK = 512
_MAXB = _N // _BK
_NBC = 15
_NBT = _NBC * _BK
_NW = 32
_GR = _NBT // _NW
_GC = 48
_CT = _N // _NW
_CC = 16


def _gate_body(q_ref, wv_ref, bv_ref, wo_ref, bo_ref, gw_ref, gb_ref,
               prob_ref, wcomb_ref, cnt_ref, imp_ref, loss_ref):
    i = pl.program_id(0)
    v = jnp.dot(q_ref[...], wv_ref[...], preferred_element_type=jnp.float32)
    v = v + bv_ref[...]
    att = jnp.dot(v, wo_ref[...], preferred_element_type=jnp.float32)
    att = att + bo_ref[...]
    logits = jnp.dot(att, gw_ref[...], preferred_element_type=jnp.float32)
    logits = logits + gb_ref[...]
    m = jnp.max(logits, axis=-1, keepdims=True)
    ex = jnp.exp(logits - m)
    probs = ex / jnp.sum(ex, axis=-1, keepdims=True)
    prob_ref[...] = probs

    lane = jax.lax.broadcasted_iota(jnp.int32, probs.shape, 1)
    p1 = jnp.max(probs, axis=-1, keepdims=True)
    i1 = jnp.argmax(probs, axis=-1)
    oh1 = lane == i1[:, None]
    masked = jnp.where(oh1, -jnp.inf, probs)
    p2 = jnp.max(masked, axis=-1, keepdims=True)
    i2 = jnp.argmax(masked, axis=-1)
    oh2 = lane == i2[:, None]
    ed = jnp.exp(p2 - p1)
    w1 = 1.0 / (1.0 + ed)
    w2 = ed / (1.0 + ed)
    wcomb_ref[...] = jnp.where(oh1, w1, 0.0) + jnp.where(oh2, w2, 0.0)

    @pl.when(i == 0)
    def _init():
        cnt_ref[...] = jnp.zeros_like(cnt_ref)
        imp_ref[...] = jnp.zeros_like(imp_ref)

    cnt_ref[...] += jnp.sum((oh1 | oh2).astype(jnp.float32), axis=0, keepdims=True)
    imp_ref[...] += jnp.sum(probs, axis=0, keepdims=True)

    @pl.when(i == pl.num_programs(0) - 1)
    def _fin():
        imp = imp_ref[0, :]
        mean = jnp.mean(imp)
        var = jnp.sum((imp - mean) ** 2) / (_E - 1)
        loss_ref[...] = jnp.broadcast_to(_W_IMP * var / (mean * mean), (1, 1))


def _route_body(cnt_ref, wc_ref, pos2_ref, ws2_ref, nblk_ref, coff_ref,
                carry_ref):
    c = pl.program_id(0)
    counts = cnt_ref[...]
    padded = jnp.ceil(counts / _BK) * _BK
    erow = jax.lax.broadcasted_iota(jnp.int32, (_E, _E), 0)
    ecol = jax.lax.broadcasted_iota(jnp.int32, (_E, _E), 1)
    offs = jnp.dot(padded, (erow < ecol).astype(jnp.float32),
                   preferred_element_type=jnp.float32)

    wc = wc_ref[...]
    msk = (wc > 0).astype(jnp.float32)
    rrow = jax.lax.broadcasted_iota(jnp.int32, (128, 128), 0)
    rcol = jax.lax.broadcasted_iota(jnp.int32, (128, 128), 1)
    tri = (rrow >= rcol).astype(jnp.float32)
    incl = jnp.dot(tri, msk, preferred_element_type=jnp.float32)
    carry = jnp.where(c == 0, jnp.zeros_like(carry_ref[...]), carry_ref[...])
    rank = incl - msk + carry
    carry_ref[...] = carry + jnp.sum(msk, axis=0, keepdims=True)

    dest = offs + rank
    dmin = jnp.where(msk > 0, dest, 1e9)
    pos_a = jnp.min(dmin, axis=1, keepdims=True)
    dmax = jnp.where(msk > 0, dest, -1.0)
    pos_b = jnp.max(dmax, axis=1, keepdims=True)
    pos2_ref[...] = jnp.concatenate([pos_a, pos_b], axis=1).astype(jnp.int32)
    w_a = jnp.sum(jnp.where(dmin == pos_a, wc, 0.0), axis=1, keepdims=True)
    w_b = jnp.sum(jnp.where(dmax == pos_b, wc, 0.0), axis=1, keepdims=True)
    ws2_ref[...] = jnp.concatenate([w_a, w_b], axis=1)

    @pl.when(c == 0)
    def _blk():
        nblk_ref[...] = (padded / _BK).astype(jnp.int32)
        coff_ref[...] = (offs / _BK).astype(jnp.int32)


def _gmm_body(nb_ref, co_ref, xs_ref, ew1_ref, eb1_ref, ew2_ref, eb2_ref,
              ws_ref, z_ref):
    e = pl.program_id(0)
    j = pl.program_id(1)

    @pl.when(j < nb_ref[e])
    def _():
        h = jnp.dot(xs_ref[...], ew1_ref[0], preferred_element_type=jnp.float32)
        h = jnp.maximum(h + eb1_ref[0], 0.0)
        out = jnp.dot(h, ew2_ref[0], preferred_element_type=jnp.float32)
        z_ref[...] = (out + eb2_ref[0]) * ws_ref[...]


def _wid():
    return lax.axis_index("s") * 2 + lax.axis_index("c")


@functools.cache
def _sc_kernels():
    mesh = plsc.VectorSubcoreMesh(core_axis_name="c", subcore_axis_name="s")
    params = pltpu.CompilerParams(needs_layout_passes=False)

    @functools.partial(
        pl.kernel,
        out_type=(jax.ShapeDtypeStruct((_NBT,), jnp.int32),
                  jax.ShapeDtypeStruct((_NBT,), jnp.float32)),
        mesh=mesh,
        scratch_types=[
            pltpu.VMEM((_N,), jnp.int32),
            pltpu.VMEM((_N,), jnp.int32),
            pltpu.VMEM((_N,), jnp.float32),
            pltpu.VMEM((_N,), jnp.float32),
            pltpu.VMEM((_NBT,), jnp.int32),
            pltpu.VMEM((_NBT,), jnp.float32),
        ],
        compiler_params=params,
    )
    def sc_scatter(pa_hbm, pb_hbm, wa_hbm, wb_hbm, idx_hbm, wso_hbm,
                   pa_v, pb_v, wa_v, wb_v, idx_v, w_v):
        @pl.when(_wid() == 0)
        def _():
            pltpu.sync_copy(pa_hbm, pa_v)
            pltpu.sync_copy(pb_hbm, pb_v)
            pltpu.sync_copy(wa_hbm, wa_v)
            pltpu.sync_copy(wb_hbm, wb_v)

            def zero_body(i, _):
                idx_v[pl.ds(i * 16, 16)] = jnp.zeros((16,), jnp.int32)
                w_v[pl.ds(i * 16, 16)] = jnp.zeros((16,), jnp.float32)
                return 0

            lax.fori_loop(0, _NBT // 16, zero_body, 0)

            def scat_body(i, _):
                t = i * 16 + lax.iota(jnp.int32, 16)
                pa = pa_v[pl.ds(i * 16, 16)]
                plsc.store_scatter(idx_v, [pa], t)
                plsc.store_scatter(w_v, [pa], wa_v[pl.ds(i * 16, 16)])
                pb = pb_v[pl.ds(i * 16, 16)]
                plsc.store_scatter(idx_v, [pb], t)
                plsc.store_scatter(w_v, [pb], wb_v[pl.ds(i * 16, 16)])
                return 0

            lax.fori_loop(0, _N // 16, scat_body, 0)
            pltpu.sync_copy(idx_v, idx_hbm)
            pltpu.sync_copy(w_v, wso_hbm)

    @functools.partial(
        pl.kernel,
        out_type=jax.ShapeDtypeStruct((_NBT, _EMB), jnp.float32),
        mesh=mesh,
        scratch_types=[
            pltpu.VMEM((_GR,), jnp.int32),
            pltpu.VMEM((_GC, _EMB), jnp.float32),
            pltpu.VMEM((_GC, _EMB), jnp.float32),
            pltpu.SemaphoreType.DMA,
            pltpu.SemaphoreType.DMA,
            pltpu.SemaphoreType.DMA,
            pltpu.SemaphoreType.DMA,
        ],
        compiler_params=params,
    )
    def sc_gather(idx_hbm, xf_hbm, xs_hbm, idx_v, buf0, buf1,
                  gs0, gs1, ws0, ws1):
        base = _wid() * _GR
        pltpu.sync_copy(idx_hbm.at[pl.ds(base, _GR)], idx_v)
        bufs = (buf0, buf1)
        gsems = (gs0, gs1)
        wsems = (ws0, ws1)
        prev_w = [None, None]
        for c in range(_GR // _GC):
            b = c & 1
            if prev_w[b] is not None:
                prev_w[b].wait()
            pltpu.async_copy(xf_hbm.at[idx_v.at[pl.ds(c * _GC, _GC)]],
                             bufs[b], gsems[b]).wait()
            prev_w[b] = pltpu.async_copy(
                bufs[b], xs_hbm.at[pl.ds(base + c * _GC, _GC)], wsems[b])
        for b in range(2):
            if prev_w[b] is not None:
                prev_w[b].wait()

    @functools.partial(
        pl.kernel,
        out_type=jax.ShapeDtypeStruct((_N, _EMB), jnp.float32),
        mesh=mesh,
        scratch_types=[
            pltpu.VMEM((_CT,), jnp.int32),
            pltpu.VMEM((_CT,), jnp.int32),
            pltpu.VMEM((_CC, _EMB), jnp.float32),
            pltpu.VMEM((_CC, _EMB), jnp.float32),
            pltpu.VMEM((_CC, _EMB), jnp.float32),
            pltpu.VMEM((_CC, _EMB), jnp.float32),
            pltpu.VMEM((_CC, _EMB), jnp.float32),
            pltpu.VMEM((_CC, _EMB), jnp.float32),
            pltpu.SemaphoreType.DMA,
            pltpu.SemaphoreType.DMA,
            pltpu.SemaphoreType.DMA,
            pltpu.SemaphoreType.DMA,
            pltpu.SemaphoreType.DMA,
            pltpu.SemaphoreType.DMA,
        ],
        compiler_params=params,
    )
    def sc_combine(pa_hbm, pb_hbm, z_hbm, y_hbm,
                   pa_v, pb_v, a0, a1, b0, b1, o0, o1,
                   sa0, sa1, sb0, sb1, so0, so1):
        base = _wid() * _CT
        pltpu.sync_copy(pa_hbm.at[pl.ds(base, _CT)], pa_v)
        pltpu.sync_copy(pb_hbm.at[pl.ds(base, _CT)], pb_v)
        abufs = (a0, a1)
        bbufs = (b0, b1)
        obufs = (o0, o1)
        sas = (sa0, sa1)
        sbs = (sb0, sb1)
        sos = (so0, so1)
        nch = _CT // _CC
        ga = [None] * nch
        gb = [None] * nch
        wout = [None, None]
        ga[0] = pltpu.async_copy(z_hbm.at[pa_v.at[pl.ds(0, _CC)]], abufs[0], sas[0])
        gb[0] = pltpu.async_copy(z_hbm.at[pb_v.at[pl.ds(0, _CC)]], bbufs[0], sbs[0])
        for c in range(nch):
            b = c & 1
            ga[c].wait()
            gb[c].wait()
            if c + 1 < nch:
                nb = (c + 1) & 1
                ga[c + 1] = pltpu.async_copy(
                    z_hbm.at[pa_v.at[pl.ds((c + 1) * _CC, _CC)]], abufs[nb], sas[nb])
                gb[c + 1] = pltpu.async_copy(
                    z_hbm.at[pb_v.at[pl.ds((c + 1) * _CC, _CC)]], bbufs[nb], sbs[nb])
            if wout[b] is not None:
                wout[b].wait()
            ra, rb, ro = abufs[b], bbufs[b], obufs[b]

            @plsc.parallel_loop(0, _CC * (_EMB // 16), unroll=8)
            def _add(i):
                j = i // (_EMB // 16)
                k = (i % (_EMB // 16)) * 16
                ro[j, pl.ds(k, 16)] = ra[j, pl.ds(k, 16)] + rb[j, pl.ds(k, 16)]

            wout[b] = pltpu.async_copy(
                obufs[b], y_hbm.at[pl.ds(base + c * _CC, _CC)], sos[b])
        for b in range(2):
            if wout[b] is not None:
                wout[b].wait()

    return sc_scatter, sc_gather, sc_combine


def kernel(x, q, Wq, bq, Wk, bk, Wv, bv, Wo, bo, gate_W, gate_b, ew1, eb1, ew2, eb2):
    x_shape = x.shape
    xf = x.reshape(-1, x_shape[-1])
    tb = 512
    grid_t = _N // tb

    probs, wcomb, counts, _imp, loss = pl.pallas_call(
        _gate_body,
        grid=(grid_t,),
        in_specs=[
            pl.BlockSpec((tb, _EMB), lambda i: (i, 0)),
            pl.BlockSpec((_EMB, _EMB), lambda i: (0, 0)),
            pl.BlockSpec((1, _EMB), lambda i: (0, 0)),
            pl.BlockSpec((_EMB, _EMB), lambda i: (0, 0)),
            pl.BlockSpec((1, _EMB), lambda i: (0, 0)),
            pl.BlockSpec((_EMB, _E), lambda i: (0, 0)),
            pl.BlockSpec((1, _E), lambda i: (0, 0)),
        ],
        out_specs=[
            pl.BlockSpec((tb, _E), lambda i: (i, 0)),
            pl.BlockSpec((tb, _E), lambda i: (i, 0)),
            pl.BlockSpec((1, _E), lambda i: (0, 0)),
            pl.BlockSpec((1, _E), lambda i: (0, 0)),
            pl.BlockSpec((1, 1), lambda i: (0, 0)),
        ],
        out_shape=[
            jax.ShapeDtypeStruct((_N, _E), jnp.float32),
            jax.ShapeDtypeStruct((_N, _E), jnp.float32),
            jax.ShapeDtypeStruct((1, _E), jnp.float32),
            jax.ShapeDtypeStruct((1, _E), jnp.float32),
            jax.ShapeDtypeStruct((1, 1), jnp.float32),
        ],
    )(q, Wv, bv.reshape(1, _EMB), Wo, bo.reshape(1, _EMB),
      gate_W, gate_b.reshape(1, _E))

    pos2, ws2, nblk, coff = pl.pallas_call(
        _route_body,
        grid=(_N // 128,),
        in_specs=[
            pl.BlockSpec((1, _E), lambda c: (0, 0)),
            pl.BlockSpec((128, _E), lambda c: (c, 0)),
        ],
        out_specs=[
            pl.BlockSpec((128, 2), lambda c: (c, 0)),
            pl.BlockSpec((128, 2), lambda c: (c, 0)),
            pl.BlockSpec((1, _E), lambda c: (0, 0)),
            pl.BlockSpec((1, _E), lambda c: (0, 0)),
        ],
        out_shape=[
            jax.ShapeDtypeStruct((_N, 2), jnp.int32),
            jax.ShapeDtypeStruct((_N, 2), jnp.float32),
            jax.ShapeDtypeStruct((1, _E), jnp.int32),
            jax.ShapeDtypeStruct((1, _E), jnp.int32),
        ],
        scratch_shapes=[pltpu.VMEM((1, _E), jnp.float32)],
    )(counts, wcomb)

    sc_scatter, sc_gather, sc_combine = _sc_kernels()

    pos_a = pos2[:, 0]
    pos_b = pos2[:, 1]
    idx_sorted, w_sorted = sc_scatter(pos_a, pos_b, ws2[:, 0], ws2[:, 1])
    xs = sc_gather(idx_sorted, xf)

    def _blk(j, nb, co):
        return co + jnp.maximum(jnp.minimum(j, nb - 1), 0)

    grid_spec = pltpu.PrefetchScalarGridSpec(
        num_scalar_prefetch=2,
        grid=(_E, _MAXB),
        in_specs=[
            pl.BlockSpec((_BK, _EMB), lambda e, j, nb, co: (_blk(j, nb[e], co[e]), 0)),
            pl.BlockSpec((1, _EMB, _EMB), lambda e, j, nb, co: (e, 0, 0)),
            pl.BlockSpec((1, 1, _EMB), lambda e, j, nb, co: (e, 0, 0)),
            pl.BlockSpec((1, _EMB, _EMB), lambda e, j, nb, co: (e, 0, 0)),
            pl.BlockSpec((1, 1, _EMB), lambda e, j, nb, co: (e, 0, 0)),
            pl.BlockSpec((_BK, 1), lambda e, j, nb, co: (_blk(j, nb[e], co[e]), 0)),
        ],
        out_specs=pl.BlockSpec(
            (_BK, _EMB), lambda e, j, nb, co: (_blk(j, nb[e], co[e]), 0)),
    )
    z = pl.pallas_call(
        _gmm_body,
        grid_spec=grid_spec,
        out_shape=jax.ShapeDtypeStruct((_NBT, _EMB), jnp.float32),
    )(nblk.reshape(_E), coff.reshape(_E), xs, ew1, eb1.reshape(_E, 1, _EMB),
      ew2, eb2.reshape(_E, 1, _EMB), w_sorted.reshape(_NBT, 1))

    y = sc_combine(pos_a, pos_b, z)
    return (y.reshape(x_shape), probs, loss.reshape(()))


# speedup vs baseline: 2.9084x; 2.9084x over previous
import jax
import jax.numpy as jnp
from jax.experimental import pallas as pl
from jax.experimental.pallas import tpu as pltpu

_EMB = 1024
_E = 8
_TOP = 2
_W_IMP = 0.01


def _gate_body(q_ref, wv_ref, bv_ref, wo_ref, bo_ref, gw_ref, gb_ref,
               prob_ref, wcomb_ref, imp_ref, loss_ref):
    i = pl.program_id(0)
    v = jnp.dot(q_ref[...], wv_ref[...], preferred_element_type=jnp.float32)
    v = v + bv_ref[...]
    att = jnp.dot(v, wo_ref[...], preferred_element_type=jnp.float32)
    att = att + bo_ref[...]
    logits = jnp.dot(att, gw_ref[...], preferred_element_type=jnp.float32)
    logits = logits + gb_ref[...]
    m = jnp.max(logits, axis=-1, keepdims=True)
    ex = jnp.exp(logits - m)
    probs = ex / jnp.sum(ex, axis=-1, keepdims=True)
    prob_ref[...] = probs

    lane = jax.lax.broadcasted_iota(jnp.int32, probs.shape, 1)
    p1 = jnp.max(probs, axis=-1, keepdims=True)
    i1 = jnp.argmax(probs, axis=-1)
    oh1 = lane == i1[:, None]
    masked = jnp.where(oh1, -jnp.inf, probs)
    p2 = jnp.max(masked, axis=-1, keepdims=True)
    i2 = jnp.argmax(masked, axis=-1)
    oh2 = lane == i2[:, None]
    ed = jnp.exp(p2 - p1)
    w1 = 1.0 / (1.0 + ed)
    w2 = ed / (1.0 + ed)
    wcomb_ref[...] = jnp.where(oh1, w1, 0.0) + jnp.where(oh2, w2, 0.0)

    @pl.when(i == 0)
    def _init():
        imp_ref[...] = jnp.zeros_like(imp_ref)

    imp_ref[...] += jnp.sum(probs, axis=0, keepdims=True)

    @pl.when(i == pl.num_programs(0) - 1)
    def _fin():
        imp = imp_ref[0, :]
        mean = jnp.mean(imp)
        var = jnp.sum((imp - mean) ** 2) / (_E - 1)
        loss_ref[...] = jnp.broadcast_to(_W_IMP * var / (mean * mean), (1, 1))


def _expert_body(xf_ref, ew1_ref, eb1_ref, ew2_ref, eb2_ref, wc_ref, y_ref):
    e = pl.program_id(0)
    x = xf_ref[...]
    h = jnp.dot(x, ew1_ref[0], preferred_element_type=jnp.float32)
    h = jnp.maximum(h + eb1_ref[0], 0.0)
    out = jnp.dot(h, ew2_ref[0], preferred_element_type=jnp.float32)
    out = out + eb2_ref[0]
    onehot = (jax.lax.broadcasted_iota(jnp.int32, (_E, 1), 0) == e).astype(jnp.float32)
    w = jnp.dot(wc_ref[...], onehot, preferred_element_type=jnp.float32)

    @pl.when(e == 0)
    def _init():
        y_ref[...] = jnp.zeros_like(y_ref)

    y_ref[...] += out * w


def kernel(x, q, Wq, bq, Wk, bk, Wv, bv, Wo, bo, gate_W, gate_b, ew1, eb1, ew2, eb2):
    x_shape = x.shape
    xf = x.reshape(-1, x_shape[-1])
    n = xf.shape[0]
    tb = 512
    grid_t = n // tb

    probs, wcomb, _imp, loss = pl.pallas_call(
        _gate_body,
        grid=(grid_t,),
        in_specs=[
            pl.BlockSpec((tb, _EMB), lambda i: (i, 0)),
            pl.BlockSpec((_EMB, _EMB), lambda i: (0, 0)),
            pl.BlockSpec((1, _EMB), lambda i: (0, 0)),
            pl.BlockSpec((_EMB, _EMB), lambda i: (0, 0)),
            pl.BlockSpec((1, _EMB), lambda i: (0, 0)),
            pl.BlockSpec((_EMB, _E), lambda i: (0, 0)),
            pl.BlockSpec((1, _E), lambda i: (0, 0)),
        ],
        out_specs=[
            pl.BlockSpec((tb, _E), lambda i: (i, 0)),
            pl.BlockSpec((tb, _E), lambda i: (i, 0)),
            pl.BlockSpec((1, _E), lambda i: (0, 0)),
            pl.BlockSpec((1, 1), lambda i: (0, 0)),
        ],
        out_shape=[
            jax.ShapeDtypeStruct((n, _E), jnp.float32),
            jax.ShapeDtypeStruct((n, _E), jnp.float32),
            jax.ShapeDtypeStruct((1, _E), jnp.float32),
            jax.ShapeDtypeStruct((1, 1), jnp.float32),
        ],
    )(q, Wv, bv.reshape(1, _EMB), Wo, bo.reshape(1, _EMB),
      gate_W, gate_b.reshape(1, _E))

    y = pl.pallas_call(
        _expert_body,
        grid=(_E,),
        in_specs=[
            pl.BlockSpec((n, _EMB), lambda e: (0, 0)),
            pl.BlockSpec((1, _EMB, _EMB), lambda e: (e, 0, 0)),
            pl.BlockSpec((1, 1, _EMB), lambda e: (e, 0, 0)),
            pl.BlockSpec((1, _EMB, _EMB), lambda e: (e, 0, 0)),
            pl.BlockSpec((1, 1, _EMB), lambda e: (e, 0, 0)),
            pl.BlockSpec((n, _E), lambda e: (0, 0)),
        ],
        out_specs=pl.BlockSpec((n, _EMB), lambda e: (0, 0)),
        out_shape=jax.ShapeDtypeStruct((n, _EMB), jnp.float32),
    )(xf, ew1, eb1.reshape(_E, 1, _EMB), ew2, eb2.reshape(_E, 1, _EMB), wcomb)

    return (y.reshape(x_shape), probs, loss.reshape(()))
